# baseline (speedup 1.0000x reference)
import functools

import jax
import jax.numpy as jnp
from jax import lax
from jax.experimental import pallas as pl
from jax.experimental.pallas import tpu as pltpu
from jax.experimental.pallas import tpu_sc as plsc

N_NUM = 13
N_CAT = 26
VOCAB = 100000
D = 64
B = 4096
L = 16
NC = 2
NS = 16
NW = NC * NS
ROWS_PER_W = B // NW
CH = 8
CPT = ROWS_PER_W // CH
N_TOK = 1 + N_NUM + N_CAT


def _sc_tokenizer(xnum_hbm, fidx_hbm, w_hbm, bias_hbm, table_hbm, cls_hbm,
                  out_hbm, idx_v, comb, xnum_v, w_v, b_v, cls_v, gsem, osem):
    wid = lax.axis_index("s") * NC + lax.axis_index("c")
    base_row = wid * ROWS_PER_W

    pltpu.sync_copy(fidx_hbm.at[pl.ds(base_row, ROWS_PER_W)], idx_v)
    pltpu.sync_copy(xnum_hbm.at[pl.ds(base_row, ROWS_PER_W)], xnum_v)
    pltpu.sync_copy(w_hbm, w_v)
    pltpu.sync_copy(bias_hbm, b_v)
    pltpu.sync_copy(cls_hbm, cls_v)

    def chunk_body(c, carry):
        gathers = []
        for b in range(CH):
            gathers.append(pltpu.async_copy(
                table_hbm.at[idx_v.at[c * CH + b]],
                comb.at[b, pl.ds(1 + N_NUM, N_CAT)],
                gsem))
        for b in range(CH):
            for k in range(D // L):
                comb[b, 0, pl.ds(k * L, L)] = cls_v[pl.ds(k * L, L)]
        for b in range(CH):
            xrow = xnum_v[c * CH + b, pl.ds(0, L)]
            for j in range(N_NUM):
                sv = jnp.full((L,), xrow[j], dtype=jnp.float32)
                for k in range(D // L):
                    comb[b, 1 + j, pl.ds(k * L, L)] = (
                        sv * w_v[j, pl.ds(k * L, L)] + b_v[j, pl.ds(k * L, L)])
        for g in gathers:
            g.wait()
        pltpu.async_copy(
            comb, out_hbm.at[pl.ds(base_row + c * CH, CH)], osem).wait()
        return carry

    lax.fori_loop(0, CPT, chunk_body, 0)


@jax.jit
def _tokenize(x_num_p, fidx, num_weight, num_bias, table_flat, cls_flat):
    mesh = plsc.VectorSubcoreMesh(core_axis_name="c", subcore_axis_name="s")
    run = functools.partial(
        pl.kernel,
        out_type=jax.ShapeDtypeStruct((B, N_TOK, D), jnp.float32),
        mesh=mesh,
        compiler_params=pltpu.CompilerParams(use_tc_tiling_on_sc=False),
        scratch_types=[
            pltpu.VMEM((ROWS_PER_W, N_CAT), jnp.int32),
            pltpu.VMEM((CH, N_TOK, D), jnp.float32),
            pltpu.VMEM((ROWS_PER_W, 16), jnp.float32),
            pltpu.VMEM((N_NUM, D), jnp.float32),
            pltpu.VMEM((N_NUM, D), jnp.float32),
            pltpu.VMEM((D,), jnp.float32),
            pltpu.SemaphoreType.DMA,
            pltpu.SemaphoreType.DMA,
        ],
    )(_sc_tokenizer)
    return run(x_num_p, fidx, num_weight, num_bias, table_flat, cls_flat)


def kernel(x_num, x_cat, num_weight, num_bias, cat_tables, cls_token):
    fidx = x_cat.astype(jnp.int32) + (
        jnp.arange(N_CAT, dtype=jnp.int32) * VOCAB)[None, :]
    table_flat = cat_tables.reshape(N_CAT * VOCAB, D)
    x_num_p = jnp.pad(x_num, ((0, 0), (0, 16 - N_NUM)))
    cls_flat = cls_token.reshape(D)
    return _tokenize(x_num_p, fidx, num_weight, num_bias, table_flat, cls_flat)
